# baseline (speedup 1.0000x reference)
import functools

import jax
import jax.numpy as jnp
from jax import lax
from jax.experimental import pallas as pl
from jax.experimental.pallas import tpu as pltpu
from jax.experimental.pallas import tpu_sc as plsc

_NC = 2
_NS = 16
_L = 16
_NW = _NC * _NS
_CHUNK = 128


@functools.lru_cache(maxsize=None)
def _build(n: int, c: int):
  assert n % (_NW * _CHUNK) == 0, (n, c)
  b_per_w = n // _NW
  nchunk = b_per_w // _CHUNK

  mesh = plsc.VectorSubcoreMesh(core_axis_name="c", subcore_axis_name="s")

  @functools.partial(
      pl.kernel,
      mesh=mesh,
      out_type=jax.ShapeDtypeStruct((_NW * _L,), jnp.float32),
      scratch_types=[
          pltpu.VMEM((b_per_w,), jnp.int32),
          pltpu.VMEM((nchunk, _CHUNK), jnp.int32),
          pltpu.VMEM((nchunk, _CHUNK), jnp.float32),
          pltpu.VMEM((_L,), jnp.float32),
          pltpu.SemaphoreType.DMA,
      ],
  )
  def ppd(flat_hbm, tgt_hbm, out_hbm, tgt_v, idx_v, val_v, acc_v, sem):
    wid = lax.axis_index("s") * _NC + lax.axis_index("c")
    base = wid * b_per_w
    pltpu.sync_copy(tgt_hbm.at[pl.ds(base, b_per_w)], tgt_v)
    base_flat = base * c
    for ch in range(nchunk):
      for i in range(_CHUNK // _L):
        j0 = ch * _CHUNK + i * _L
        t = tgt_v[pl.ds(j0, _L)]
        rows = base_flat + (lax.iota(jnp.int32, _L) + j0) * c
        idx_v[ch, pl.ds(i * _L, _L)] = rows + t
    copies = [
        pltpu.async_copy(flat_hbm.at[idx_v.at[ch]], val_v.at[ch], sem)
        for ch in range(nchunk)
    ]
    for cp in copies:
      cp.wait()
    acc = jnp.zeros((_L,), jnp.float32)
    for ch in range(nchunk):
      for i in range(_CHUNK // _L):
        v = val_v[ch, pl.ds(i * _L, _L)]
        d = 1.0 - v
        acc = acc + d * d
    acc_v[...] = acc
    pltpu.sync_copy(acc_v, out_hbm.at[pl.ds(wid * _L, _L)])

  return ppd


def kernel(contrast_logits, contrast_target):
  n, c = contrast_logits.shape
  flat = contrast_logits.reshape(-1)
  tgt = contrast_target.astype(jnp.int32)
  partials = _build(n, c)(flat, tgt)
  return jnp.sum(partials) / jnp.float32(n)


# speedup vs baseline: 8.3797x; 8.3797x over previous
import functools

import jax
import jax.numpy as jnp
from jax import lax
from jax.experimental import pallas as pl
from jax.experimental.pallas import tpu as pltpu
from jax.experimental.pallas import tpu_sc as plsc

_NC = 2
_NS = 16
_L = 16
_NW = _NC * _NS
_CHUNK = 128


@functools.lru_cache(maxsize=None)
def _build(n: int, c: int):
  assert n % (_NW * _CHUNK) == 0, (n, c)
  b_per_w = n // _NW
  nchunk = b_per_w // _CHUNK

  mesh = plsc.VectorSubcoreMesh(core_axis_name="c", subcore_axis_name="s")

  @functools.partial(
      pl.kernel,
      mesh=mesh,
      out_type=jax.ShapeDtypeStruct((_NW * _L,), jnp.float32),
      scratch_types=[
          pltpu.VMEM((b_per_w,), jnp.int32),
          pltpu.VMEM((nchunk, _CHUNK), jnp.int32),
          pltpu.VMEM((nchunk, _CHUNK), jnp.float32),
          pltpu.VMEM((_L,), jnp.float32),
          pltpu.SemaphoreType.DMA,
      ],
  )
  def ppd(flat_hbm, tgt_hbm, out_hbm, tgt_v, idx_v, val_v, acc_v, sem):
    wid = lax.axis_index("s") * _NC + lax.axis_index("c")
    base = wid * b_per_w
    pltpu.sync_copy(tgt_hbm.at[pl.ds(base, b_per_w)], tgt_v)
    ctiles = c // 128
    for ch in range(nchunk):
      for i in range(_CHUNK // _L):
        j0 = ch * _CHUNK + i * _L
        t = tgt_v[pl.ds(j0, _L)]
        r = base + j0 + lax.iota(jnp.int32, _L)
        fi = ((r >> 3) * ctiles + (t >> 7)) * 1024 + (r & 7) * 128 + (t & 127)
        idx_v[ch, pl.ds(i * _L, _L)] = fi
    copies = [
        pltpu.async_copy(flat_hbm.at[idx_v.at[ch]], val_v.at[ch], sem)
        for ch in range(nchunk)
    ]
    for cp in copies:
      cp.wait()
    acc = jnp.zeros((_L,), jnp.float32)
    for ch in range(nchunk):
      for i in range(_CHUNK // _L):
        v = val_v[ch, pl.ds(i * _L, _L)]
        d = 1.0 - v
        acc = acc + d * d
    acc_v[...] = acc
    pltpu.sync_copy(acc_v, out_hbm.at[pl.ds(wid * _L, _L)])

  return ppd


def kernel(contrast_logits, contrast_target):
  n, c = contrast_logits.shape
  flat = (
      contrast_logits.reshape(n // 8, 8, c // 128, 128)
      .transpose(0, 2, 1, 3)
      .reshape(-1)
  )
  tgt = contrast_target.astype(jnp.int32)
  partials = _build(n, c)(flat, tgt)
  return jnp.sum(partials) / jnp.float32(n)


# speedup vs baseline: 8.6768x; 1.0355x over previous
import functools

import jax
import jax.numpy as jnp
from jax import lax
from jax.experimental import pallas as pl
from jax.experimental.pallas import tpu as pltpu
from jax.experimental.pallas import tpu_sc as plsc

_NC = 2
_NS = 16
_L = 16
_NW = _NC * _NS
_CHUNK = 128


@functools.lru_cache(maxsize=None)
def _build(n: int, c: int):
  assert n % (_NW * _CHUNK) == 0, (n, c)
  b_per_w = n // _NW
  nchunk = b_per_w // _CHUNK

  mesh = plsc.VectorSubcoreMesh(core_axis_name="c", subcore_axis_name="s")

  @functools.partial(
      pl.kernel,
      mesh=mesh,
      out_type=jax.ShapeDtypeStruct((_NW * _L,), jnp.float32),
      scratch_types=[
          pltpu.VMEM((b_per_w,), jnp.int32),
          pltpu.VMEM((nchunk, _CHUNK), jnp.int32),
          pltpu.VMEM((nchunk, _CHUNK), jnp.float32),
          pltpu.VMEM((_L,), jnp.float32),
          pltpu.SemaphoreType.DMA((nchunk,)),
      ],
  )
  def ppd(flat_hbm, tgt_hbm, out_hbm, tgt_v, idx_v, val_v, acc_v, sems):
    wid = lax.axis_index("s") * _NC + lax.axis_index("c")
    base = wid * b_per_w
    pltpu.sync_copy(tgt_hbm.at[pl.ds(base, b_per_w)], tgt_v)
    ctiles = c // 128
    iota = lax.iota(jnp.int32, _L)
    kvec = (iota >> 3) * (ctiles * 1024) + (iota & 7) * 128
    copies = []
    for ch in range(nchunk):
      for i in range(_CHUNK // _L):
        j0 = ch * _CHUNK + i * _L
        t = tgt_v[pl.ds(j0, _L)]
        s = ((base + j0) >> 3) * (ctiles * 1024)
        fi = s + kvec + (t >> 7) * 1024 + (t & 127)
        idx_v[ch, pl.ds(i * _L, _L)] = fi
      copies.append(
          pltpu.async_copy(flat_hbm.at[idx_v.at[ch]], val_v.at[ch], sems.at[ch])
      )
    accs = [jnp.zeros((_L,), jnp.float32)] * (_CHUNK // _L)
    for ch in range(nchunk):
      copies[ch].wait()
      for i in range(_CHUNK // _L):
        v = val_v[ch, pl.ds(i * _L, _L)]
        d = 1.0 - v
        accs[i] = accs[i] + d * d
    while len(accs) > 1:
      accs = [a + b for a, b in zip(accs[::2], accs[1::2])]
    acc_v[...] = accs[0]
    pltpu.sync_copy(acc_v, out_hbm.at[pl.ds(wid * _L, _L)])

  return ppd


def kernel(contrast_logits, contrast_target):
  n, c = contrast_logits.shape
  flat = (
      contrast_logits.reshape(n // 8, 8, c // 128, 128)
      .transpose(0, 2, 1, 3)
      .reshape(-1)
  )
  tgt = contrast_target.astype(jnp.int32)
  partials = _build(n, c)(flat, tgt)
  return jnp.sum(partials) / jnp.float32(n)


# speedup vs baseline: 9.2390x; 1.0648x over previous
import functools

import jax
import jax.numpy as jnp
from jax import lax
from jax.experimental import pallas as pl
from jax.experimental.pallas import tpu as pltpu
from jax.experimental.pallas import tpu_sc as plsc

_NS = 16
_L = 16
_CHUNK = 128


@functools.lru_cache(maxsize=None)
def _build(n: int, c: int):
  assert n % (_NS * _CHUNK) == 0 and c % 128 == 0, (n, c)
  b_per_w = n // _NS
  nchunk = b_per_w // _CHUNK

  mesh = plsc.VectorSubcoreMesh(
      core_axis_name="c", subcore_axis_name="s", num_cores=1)

  @functools.partial(
      pl.kernel,
      mesh=mesh,
      out_type=jax.ShapeDtypeStruct((1,), jnp.float32),
      scratch_types=[
          pltpu.VMEM((b_per_w,), jnp.int32),
          pltpu.VMEM((nchunk, _CHUNK), jnp.int32),
          pltpu.VMEM((nchunk, _CHUNK), jnp.float32),
          pltpu.VMEM((_L,), jnp.float32),
          pltpu.VMEM((_NS, _L), jnp.float32),
          pltpu.VMEM_SHARED((_NS, _L), jnp.float32),
          pltpu.SemaphoreType.DMA((nchunk,)),
      ],
  )
  def ppd(flat_hbm, tgt_hbm, out_hbm, tgt_v, idx_v, val_v, acc_v, red_v,
          shared, sems):
    sid = lax.axis_index("s")
    base = sid * b_per_w
    pltpu.sync_copy(tgt_hbm.at[pl.ds(base, b_per_w)], tgt_v)
    ctiles = c // 128
    iota = lax.iota(jnp.int32, _L)
    kvec = (iota >> 3) * (ctiles * 1024) + (iota & 7) * 128
    copies = []
    for ch in range(nchunk):
      for i in range(_CHUNK // _L):
        j0 = ch * _CHUNK + i * _L
        t = tgt_v[pl.ds(j0, _L)]
        s = ((base + j0) >> 3) * (ctiles * 1024)
        fi = s + kvec + (t >> 7) * 1024 + (t & 127)
        idx_v[ch, pl.ds(i * _L, _L)] = fi
      copies.append(
          pltpu.async_copy(flat_hbm.at[idx_v.at[ch]], val_v.at[ch], sems.at[ch])
      )
    accs = [jnp.zeros((_L,), jnp.float32)] * (_CHUNK // _L)
    for ch in range(nchunk):
      copies[ch].wait()
      for i in range(_CHUNK // _L):
        v = val_v[ch, pl.ds(i * _L, _L)]
        d = 1.0 - v
        accs[i] = accs[i] + d * d
    while len(accs) > 1:
      accs = [a + b for a, b in zip(accs[::2], accs[1::2])]
    acc_v[...] = accs[0]
    pltpu.sync_copy(acc_v, shared.at[sid])
    plsc.subcore_barrier()

    @pl.when(sid == 0)
    def _():
      pltpu.sync_copy(shared, red_v)
      tot = red_v[0, :]
      for w in range(1, _NS):
        tot = tot + red_v[w, :]
      scaled = tot * (1.0 / n)
      s = scaled[0]
      for i in range(1, _L):
        s = s + scaled[i]
      acc_v[...] = jnp.full((_L,), s, jnp.float32)
      pltpu.sync_copy(acc_v.at[pl.ds(0, 1)], out_hbm)

  return ppd


def kernel(contrast_logits, contrast_target):
  n, c = contrast_logits.shape
  flat = (
      contrast_logits.reshape(n // 8, 8, c // 128, 128)
      .transpose(0, 2, 1, 3)
      .reshape(-1)
  )
  tgt = contrast_target.astype(jnp.int32)
  res = _build(n, c)(flat, tgt)
  return res.reshape(())
